# baseline (speedup 1.0000x reference)
import functools

import jax
import jax.numpy as jnp
from jax import lax
from jax.experimental import pallas as pl
from jax.experimental.pallas import tpu as pltpu
from jax.experimental.pallas import tpu_sc as plsc

_LANES = 16
_BI = 128
_CI = 8
_NBUF = 2


def _embed_kernel(seq_len, d, n_jb, scale):
    mesh = plsc.VectorSubcoreMesh(core_axis_name="c", subcore_axis_name="s")
    n_jc = d // _CI
    kvecs = d // _LANES

    @functools.partial(
        pl.kernel,
        mesh=mesh,
        out_type=jax.ShapeDtypeStruct((seq_len, n_jc, n_jb, _CI, _BI), jnp.float32),
        scratch_types=[
            pltpu.VMEM((seq_len, _BI), jnp.int32),
            pltpu.VMEM((seq_len, _BI), jnp.int32),
            [pltpu.VMEM((_BI, 2 * d), jnp.float32)] * _NBUF,
            [pltpu.VMEM((d, _BI + 1), jnp.float32)] * _NBUF,
            [pltpu.SemaphoreType.DMA] * _NBUF,
            [pltpu.SemaphoreType.DMA] * _NBUF,
        ],
        compiler_params=pltpu.CompilerParams(
            use_tc_tiling_on_sc=True, needs_layout_passes=False
        ),
    )
    def run(table_hbm, idx_hbm, out_hbm, idx_v, pidx_v, gb, ob, sg, so):
        cid = lax.axis_index("c")
        sid = lax.axis_index("s")
        w = sid * 2 + cid
        pltpu.sync_copy(idx_hbm.at[:, w], idx_v)

        iota = jnp.arange(_LANES, dtype=jnp.int32)

        def shift_body(i, carry):
            for k in range(_BI // _LANES):
                sl = pl.ds(k * _LANES, _LANES)
                pidx_v[i, sl] = idx_v[i, sl] >> 1
            return carry

        lax.fori_loop(0, seq_len, shift_body, 0)

        for b in range(_NBUF):
            pltpu.async_copy(table_hbm.at[pidx_v.at[b]], gb[b], sg[b])

        def group_body(p, carry):
            for b in range(_NBUF):
                j = p * _NBUF + b
                pltpu.make_async_copy(
                    table_hbm.at[pl.ds(0, _BI)], gb[b], sg[b]
                ).wait()

                @pl.when(p > 0)
                def _():
                    for jc in range(n_jc):
                        pltpu.make_async_copy(
                            ob[b].at[pl.ds(jc * _CI, _CI), pl.ds(0, _BI)],
                            out_hbm.at[0, jc, w],
                            so[b],
                        ).wait()

                @plsc.parallel_loop(0, _BI // _LANES)
                def _(g):
                    t0 = g * _LANES
                    hv = (idx_v[j, pl.ds(t0, _LANES)] & 1) * d
                    for i in range(_LANES):
                        t = t0 + i
                        half = hv[i]
                        t_vec = jnp.broadcast_to(t, (_LANES,))
                        for k in range(kvecs):
                            c_idx = iota + (k * _LANES)
                            v = gb[b][t, pl.ds(half + k * _LANES, _LANES)] * scale
                            plsc.store_scatter(ob[b], [c_idx, t_vec], v)

                for jc in range(n_jc):
                    pltpu.async_copy(
                        ob[b].at[pl.ds(jc * _CI, _CI), pl.ds(0, _BI)],
                        out_hbm.at[j, jc, w],
                        so[b],
                    )

                @pl.when(j + _NBUF < seq_len)
                def _():
                    pltpu.async_copy(
                        table_hbm.at[pidx_v.at[j + _NBUF]], gb[b], sg[b]
                    )

            return carry

        lax.fori_loop(0, seq_len // _NBUF, group_body, 0)

        for b in range(_NBUF):
            for jc in range(n_jc):
                pltpu.make_async_copy(
                    ob[b].at[pl.ds(jc * _CI, _CI), pl.ds(0, _BI)],
                    out_hbm.at[0, jc, w],
                    so[b],
                ).wait()

    return run


def kernel(table, x):
    v, d = table.shape
    bsz, seq_len = x.shape
    n_jb = bsz // _BI
    scale = float(d) ** -0.5

    table2 = table.reshape(v // 2, 2 * d)
    idx = x.T.reshape(seq_len, n_jb, _BI)
    out5 = _embed_kernel(seq_len, d, n_jb, scale)(table2, idx)
    out = out5.transpose(2, 4, 0, 1, 3).reshape(bsz, seq_len, d)
    return out


# speedup vs baseline: 1.4111x; 1.4111x over previous
import functools

import jax
import jax.numpy as jnp
from jax import lax
from jax.experimental import pallas as pl
from jax.experimental.pallas import tpu as pltpu
from jax.experimental.pallas import tpu_sc as plsc

_LANES = 16
_BI = 128
_CI = 8
_NBUF = 2
_NW = 32


def _detile_kernel(d, vocab):
    mesh = plsc.VectorSubcoreMesh(core_axis_name="c", subcore_axis_name="s")
    n_full = vocab // _BI
    tail = vocab - n_full * _BI
    tb_w = _BI + 5

    @functools.partial(
        pl.kernel,
        mesh=mesh,
        out_type=jax.ShapeDtypeStruct((vocab // 2, 2 * d), jnp.float32),
        scratch_types=[
            [pltpu.VMEM((d, tb_w), jnp.float32)] * _NBUF,
            [pltpu.VMEM((_BI // 2, 2 * d), jnp.float32)] * _NBUF,
            [pltpu.SemaphoreType.DMA] * _NBUF,
            [pltpu.SemaphoreType.DMA] * _NBUF,
        ],
        compiler_params=pltpu.CompilerParams(
            use_tc_tiling_on_sc=True, needs_layout_passes=False
        ),
    )
    def run(tab_hbm, tail_hbm, out_hbm, tb, tt, sg, so):
        cid = lax.axis_index("c")
        sid = lax.axis_index("s")
        w = sid * 2 + cid
        iota = jnp.arange(_LANES, dtype=jnp.int32)

        n_mine = (n_full - w + _NW - 1) // _NW

        def jb_of(i):
            return i * _NW + w

        for b in range(_NBUF):
            @pl.when(b < n_mine)
            def _():
                pltpu.async_copy(
                    tab_hbm.at[:, pl.ds(jb_of(b) * _BI, _BI)],
                    tb[b].at[:, pl.ds(0, _BI)],
                    sg[b],
                )

        def col_body(p, carry):
            for b in range(_NBUF):
                i = p * _NBUF + b

                @pl.when(i < n_mine)
                def _():
                    jb = jb_of(i)
                    pltpu.make_async_copy(
                        tab_hbm.at[:, pl.ds(0, _BI)],
                        tb[b].at[:, pl.ds(0, _BI)],
                        sg[b],
                    ).wait()

                    @pl.when(p > 0)
                    def _():
                        pltpu.make_async_copy(
                            tt[b], out_hbm.at[pl.ds(0, _BI // 2)], so[b]
                        ).wait()

                    @plsc.parallel_loop(0, _BI, unroll=2)
                    def _(t):
                        t_vec = jnp.broadcast_to(t, (_LANES,))
                        off = (t & 1) * d
                        pr = t >> 1
                        for k in range(d // _LANES):
                            c_idx = iota + k * _LANES
                            v = plsc.load_gather(tb[b], [c_idx, t_vec])
                            tt[b][pr, pl.ds(off + k * _LANES, _LANES)] = v

                    pltpu.async_copy(
                        tt[b], out_hbm.at[pl.ds(jb * (_BI // 2), _BI // 2)], so[b]
                    )

                    @pl.when(i + _NBUF < n_mine)
                    def _():
                        pltpu.async_copy(
                            tab_hbm.at[:, pl.ds(jb_of(i + _NBUF) * _BI, _BI)],
                            tb[b].at[:, pl.ds(0, _BI)],
                            sg[b],
                        )

            return carry

        lax.fori_loop(0, (n_mine + _NBUF - 1) // _NBUF, col_body, 0)

        for b in range(_NBUF):
            @pl.when(b < n_mine)
            def _():
                pltpu.make_async_copy(
                    tt[b], out_hbm.at[pl.ds(0, _BI // 2)], so[b]
                ).wait()

        if tail:
            @pl.when(w == _NW - 1)
            def _():
                pltpu.sync_copy(tail_hbm, tt[0].at[pl.ds(0, tail // 2)])
                pltpu.sync_copy(
                    tt[0].at[pl.ds(0, tail // 2)],
                    out_hbm.at[pl.ds(n_full * (_BI // 2), tail // 2)],
                )

    return run


def _lookup_kernel(seq_len, d, n_jb, scale):
    mesh = plsc.VectorSubcoreMesh(core_axis_name="c", subcore_axis_name="s")
    n_jc = d // _CI
    kvecs = d // _LANES

    @functools.partial(
        pl.kernel,
        mesh=mesh,
        out_type=jax.ShapeDtypeStruct((seq_len, n_jc, n_jb, _CI, _BI), jnp.float32),
        scratch_types=[
            pltpu.VMEM((seq_len, _BI), jnp.int32),
            [pltpu.VMEM((_BI, d), jnp.float32)] * _NBUF,
            [pltpu.VMEM((d, _BI + 1), jnp.float32)] * _NBUF,
            [pltpu.SemaphoreType.DMA] * _NBUF,
            [pltpu.SemaphoreType.DMA] * _NBUF,
        ],
        compiler_params=pltpu.CompilerParams(
            use_tc_tiling_on_sc=False, needs_layout_passes=False
        ),
    )
    def run(table_hbm, idx_hbm, out_hbm, idx_v, gb, ob, sg, so):
        cid = lax.axis_index("c")
        sid = lax.axis_index("s")
        w = sid * 2 + cid
        pltpu.sync_copy(idx_hbm.at[:, w], idx_v)

        iota = jnp.arange(_LANES, dtype=jnp.int32)

        for b in range(_NBUF):
            pltpu.async_copy(table_hbm.at[idx_v.at[b]], gb[b], sg[b])

        def group_body(p, carry):
            for b in range(_NBUF):
                j = p * _NBUF + b
                pltpu.make_async_copy(
                    table_hbm.at[pl.ds(0, _BI)], gb[b], sg[b]
                ).wait()

                @pl.when(p > 0)
                def _():
                    for jc in range(n_jc):
                        pltpu.make_async_copy(
                            ob[b].at[pl.ds(jc * _CI, _CI), pl.ds(0, _BI)],
                            out_hbm.at[0, jc, w],
                            so[b],
                        ).wait()

                @plsc.parallel_loop(0, _BI, unroll=2)
                def _(t):
                    t_vec = jnp.broadcast_to(t, (_LANES,))
                    for k in range(kvecs):
                        c_idx = iota + (k * _LANES)
                        v = gb[b][t, pl.ds(k * _LANES, _LANES)] * scale
                        plsc.store_scatter(ob[b], [c_idx, t_vec], v)

                for jc in range(n_jc):
                    pltpu.async_copy(
                        ob[b].at[pl.ds(jc * _CI, _CI), pl.ds(0, _BI)],
                        out_hbm.at[j, jc, w],
                        so[b],
                    )

                @pl.when(j + _NBUF < seq_len)
                def _():
                    pltpu.async_copy(
                        table_hbm.at[idx_v.at[j + _NBUF]], gb[b], sg[b]
                    )

            return carry

        lax.fori_loop(0, seq_len // _NBUF, group_body, 0)

        for b in range(_NBUF):
            for jc in range(n_jc):
                pltpu.make_async_copy(
                    ob[b].at[pl.ds(jc * _CI, _CI), pl.ds(0, _BI)],
                    out_hbm.at[0, jc, w],
                    so[b],
                ).wait()

    return run


def kernel(table, x):
    v, d = table.shape
    bsz, seq_len = x.shape
    n_jb = bsz // _BI
    scale = float(d) ** -0.5

    n_full = v // _BI
    tail_rows = table[n_full * _BI:].reshape(-1, 2 * d)
    table_lin = _detile_kernel(d, v)(table.T, tail_rows).reshape(v, d)

    idx = x.T.reshape(seq_len, n_jb, _BI)
    out5 = _lookup_kernel(seq_len, d, n_jb, scale)(table_lin, idx)
    out = out5.transpose(2, 4, 0, 1, 3).reshape(bsz, seq_len, d)
    return out
